# Initial kernel scaffold; baseline (speedup 1.0000x reference)
import functools

import jax
import jax.numpy as jnp
from jax import lax
from jax.experimental import pallas as pl
from jax.experimental.pallas import tpu as pltpu
from jax.experimental.pallas import tpu_sc as plsc

N = 10000
D = 128
E = 320000
NC, NS = 2, 16
NW = NC * NS
C = 128
CHUNKS = 79
EPT = C * CHUNKS
E_PAD = NW * EPT
ROWS_PER_TILE = N // NS


def _sc_scatter_kernel(src_x_hbm, sidx_hbm, didx_hbm, w_hbm, zeros_hbm,
                       out_hbm, sidx_v, didx_v, w_v, rows_v, acc_sh, sem):
    c = lax.axis_index("c")
    s = lax.axis_index("s")
    wid = c * NS + s
    base = wid * EPT

    pltpu.sync_copy(zeros_hbm, acc_sh.at[pl.ds(s * ROWS_PER_TILE, ROWS_PER_TILE)])
    plsc.subcore_barrier()

    def chunk_body(g, _):
        off = base + g * C
        pltpu.sync_copy(sidx_hbm.at[pl.ds(off, C)], sidx_v)
        pltpu.sync_copy(didx_hbm.at[pl.ds(off, C)], didx_v)
        pltpu.sync_copy(w_hbm.at[pl.ds(off, C)], w_v)
        pltpu.async_copy(src_x_hbm.at[sidx_v], rows_v, sem).wait()

        def row_body(r, _):
            wspl = plsc.load_gather(w_v, [lax.broadcast(r, (16,))])
            for k in range(D // 16):
                sl = pl.ds(k * 16, 16)
                rows_v[r, sl] = rows_v[r, sl] * wspl
            return 0

        lax.fori_loop(0, C, row_body, 0)
        pltpu.sync_copy(rows_v, acc_sh.at[didx_v], add=True)
        return 0

    lax.fori_loop(0, CHUNKS, chunk_body, 0)
    plsc.subcore_barrier()

    pltpu.sync_copy(
        acc_sh.at[pl.ds(s * ROWS_PER_TILE, ROWS_PER_TILE)],
        out_hbm.at[c, pl.ds(s * ROWS_PER_TILE, ROWS_PER_TILE)])


_sc_call = functools.partial(
    pl.kernel,
    out_type=jax.ShapeDtypeStruct((NC, N, D), jnp.float32),
    mesh=plsc.VectorSubcoreMesh(core_axis_name="c", subcore_axis_name="s"),
    scratch_types=[
        pltpu.VMEM((C,), jnp.int32),
        pltpu.VMEM((C,), jnp.int32),
        pltpu.VMEM((C,), jnp.float32),
        pltpu.VMEM((C, D), jnp.float32),
        pltpu.VMEM_SHARED((N, D), jnp.float32),
        pltpu.SemaphoreType.DMA,
    ],
)


def _sc_scatter(src_x, sidx, didx, w, zeros):
    return _sc_call(_sc_scatter_kernel)(src_x, sidx, didx, w, zeros)


def _combine_body(p_ref, o_ref):
    o_ref[...] = p_ref[0] + p_ref[1]


def kernel(src_x, dst_x, edge_index, edge_weight):
    pad = E_PAD - E
    sidx = jnp.concatenate(
        [edge_index[0].astype(jnp.int32), jnp.zeros((pad,), jnp.int32)])
    didx = jnp.concatenate(
        [edge_index[1].astype(jnp.int32), jnp.zeros((pad,), jnp.int32)])
    w = jnp.concatenate(
        [edge_weight[:, 0], jnp.zeros((pad,), jnp.float32)])
    zeros = jnp.zeros((ROWS_PER_TILE, D), jnp.float32)

    partials = _sc_scatter(src_x, sidx, didx, w, zeros)

    BR = 500
    return pl.pallas_call(
        _combine_body,
        out_shape=jax.ShapeDtypeStruct((N, D), jnp.float32),
        grid=(N // BR,),
        in_specs=[pl.BlockSpec((NC, BR, D), lambda i: (0, i, 0))],
        out_specs=pl.BlockSpec((BR, D), lambda i: (i, 0)),
    )(partials)


# speedup vs baseline: 2.6317x; 2.6317x over previous
import functools

import jax
import jax.numpy as jnp
from jax import lax
from jax.experimental import pallas as pl
from jax.experimental.pallas import tpu as pltpu
from jax.experimental.pallas import tpu_sc as plsc

N = 10000
D = 128
E = 320000
NC, NS = 2, 16
NW = NC * NS
C = 128
CHUNKS = 79
EPT = C * CHUNKS
E_PAD = NW * EPT
N_PAD = 10240
ROWS_PER_TILE = N_PAD // NS


def _sc_scatter_kernel(src_x_hbm, sidx_hbm, didx_hbm, w_hbm, zeros_hbm,
                       out_hbm, sidx_v, didx_v, w_v, rows_v, acc_sh, sem):
    c = lax.axis_index("c")
    s = lax.axis_index("s")
    wid = c * NS + s
    base = wid * EPT

    pltpu.sync_copy(zeros_hbm, acc_sh.at[pl.ds(s * ROWS_PER_TILE, ROWS_PER_TILE)])
    plsc.subcore_barrier()

    def chunk_body(g, _):
        off = base + g * C
        pltpu.sync_copy(sidx_hbm.at[pl.ds(off, C)], sidx_v)
        pltpu.sync_copy(didx_hbm.at[pl.ds(off, C)], didx_v)
        pltpu.sync_copy(w_hbm.at[pl.ds(off * 16, C * 16)], w_v)
        pltpu.async_copy(src_x_hbm.at[sidx_v], rows_v, sem).wait()

        def row_body(r, _):
            wspl = w_v[pl.ds(r * 16, 16)]
            for k in range(D // 16):
                sl = pl.ds(k * 16, 16)
                rows_v[r, sl] = rows_v[r, sl] * wspl
            return 0

        lax.fori_loop(0, C, row_body, 0)
        pltpu.sync_copy(rows_v, acc_sh.at[didx_v], add=True)
        return 0

    lax.fori_loop(0, CHUNKS, chunk_body, 0)
    plsc.subcore_barrier()

    pltpu.sync_copy(
        acc_sh.at[pl.ds(s * ROWS_PER_TILE, ROWS_PER_TILE)],
        out_hbm.at[c, pl.ds(s * ROWS_PER_TILE, ROWS_PER_TILE)])


_sc_call = functools.partial(
    pl.kernel,
    out_type=jax.ShapeDtypeStruct((NC, N_PAD, D), jnp.float32),
    mesh=plsc.VectorSubcoreMesh(core_axis_name="c", subcore_axis_name="s"),
    scratch_types=[
        pltpu.VMEM((C,), jnp.int32),
        pltpu.VMEM((C,), jnp.int32),
        pltpu.VMEM((C * 16,), jnp.float32),
        pltpu.VMEM((C, D), jnp.float32),
        pltpu.VMEM_SHARED((N_PAD, D), jnp.float32),
        pltpu.SemaphoreType.DMA,
    ],
)


def _sc_scatter(src_x, sidx, didx, w, zeros):
    return _sc_call(_sc_scatter_kernel)(src_x, sidx, didx, w, zeros)


def _combine_body(p_ref, o_ref):
    o_ref[...] = p_ref[0] + p_ref[1]


def kernel(src_x, dst_x, edge_index, edge_weight):
    pad = E_PAD - E
    sidx = jnp.concatenate(
        [edge_index[0].astype(jnp.int32), jnp.zeros((pad,), jnp.int32)])
    didx = jnp.concatenate(
        [edge_index[1].astype(jnp.int32), jnp.zeros((pad,), jnp.int32)])
    w = jnp.repeat(jnp.concatenate(
        [edge_weight[:, 0], jnp.zeros((pad,), jnp.float32)]), 16)
    zeros = jnp.zeros((ROWS_PER_TILE, D), jnp.float32)

    partials = _sc_scatter(src_x, sidx, didx, w, zeros)

    BR = 1000
    return pl.pallas_call(
        _combine_body,
        out_shape=jax.ShapeDtypeStruct((N, D), jnp.float32),
        grid=(N // BR,),
        in_specs=[pl.BlockSpec((NC, BR, D), lambda i: (0, i, 0))],
        out_specs=pl.BlockSpec((BR, D), lambda i: (i, 0)),
    )(partials)
